# baseline (speedup 1.0000x reference)
import functools

import jax
import jax.numpy as jnp
from jax import lax
from jax.experimental import pallas as pl
from jax.experimental.pallas import tpu as pltpu
from jax.experimental.pallas import tpu_sc as plsc

N_NODES = 10000
N_EDGES = 320000
D = 128

NC = 2
NS = 16
NW = NC * NS
NPAD = 10240
R = NPAD // NW
NEP = 327680
RCH = 128
NCH = NEP // (RCH * 128)
BK = 192
FLUSH_AT = 64
NEG = float("-inf")


def _ab_body(x_ref, p_ref, wx_ref, wp_ref, b_ref, a_ref, bout_ref):
    pb = jnp.dot(p_ref[:], wp_ref[:], preferred_element_type=jnp.float32)
    a_ref[:] = (
        jnp.dot(x_ref[:], wx_ref[:], preferred_element_type=jnp.float32)
        + pb
        + b_ref[:]
    )
    bout_ref[:] = pb


def _compute_ab(xp, pp, wx, wpp, b2):
    blk = 1280
    grid = NPAD // blk
    return pl.pallas_call(
        _ab_body,
        grid=(grid,),
        in_specs=[
            pl.BlockSpec((blk, D), lambda i: (i, 0)),
            pl.BlockSpec((blk, 8), lambda i: (i, 0)),
            pl.BlockSpec((D, D), lambda i: (0, 0)),
            pl.BlockSpec((8, D), lambda i: (0, 0)),
            pl.BlockSpec((1, D), lambda i: (0, 0)),
        ],
        out_specs=[
            pl.BlockSpec((blk, D), lambda i: (i, 0)),
            pl.BlockSpec((blk, D), lambda i: (i, 0)),
        ],
        out_shape=[
            jax.ShapeDtypeStruct((NPAD, D), jnp.float32),
            jax.ShapeDtypeStruct((NPAD, D), jnp.float32),
        ],
    )(xp, pp, wx, wpp, b2)


def _sc_body(a_hbm, b_hbm, key_hbm, out_hbm,
             m_v, keych, sbuf2, dbuf2, rows2, semg, seme):
    cid = lax.axis_index("c")
    sid = lax.axis_index("s")
    wid = sid * NC + cid
    lo = wid * R
    lo_v = jnp.zeros((16,), jnp.int32) + lo

    neg = jnp.full((16,), NEG, jnp.float32)

    def init_row(i, _):
        for f in range(D // 16):
            m_v[i, f * 16:(f + 1) * 16] = neg
        return 0
    lax.fori_loop(0, R + 1, init_row, 0)

    zv = jnp.zeros((16,), jnp.int32)
    dumpv = jnp.full((16,), R, jnp.int32)
    for k16 in range(2 * BK // 16):
        sl = pl.ds(k16 * 16, 16)
        sbuf2[sl] = zv
        dbuf2[sl] = dumpv

    def start_gather(f):
        pltpu.async_copy(a_hbm.at[sbuf2.at[pl.ds(f * BK, BK)]],
                         rows2.at[pl.ds(f * BK, BK)], semg)

    def wait_gather(f):
        pltpu.make_async_copy(a_hbm.at[sbuf2.at[pl.ds(f * BK, BK)]],
                              rows2.at[pl.ds(f * BK, BK)], semg).wait()

    def drain(par, fill):
        def d16(k16, _):
            dvec = dbuf2[pl.ds(par * BK + k16 * 16, 16)]
            for j in range(16):
                r = dvec[j]
                k = k16 * 16 + j
                for f in range(D // 16):
                    sl = pl.ds(f * 16, 16)
                    m_v[r, sl] = jnp.maximum(m_v[r, sl],
                                             rows2[par * BK + k, sl])
            return 0
        lax.fori_loop(0, (fill + 15) >> 4, d16, 0)

    start_gather(jnp.int32(1))

    def start_edges(c, par):
        pltpu.async_copy(key_hbm.at[pl.ds(c * RCH, RCH)],
                         keych.at[pl.ds(par * RCH, RCH)], seme)

    def wait_edges(par):
        pltpu.make_async_copy(key_hbm.at[pl.ds(0, RCH)],
                              keych.at[pl.ds(par * RCH, RCH)], seme).wait()

    start_edges(jnp.int32(0), jnp.int32(0))

    def chunk(c, carry):
        par = c & 1
        wait_edges(par)
        start_edges(jnp.minimum(c + 1, NCH - 1), 1 - par)

        def row(rw, cr2):
            ptr, fp, pfill = cr2
            for q in range(8):
                kv = keych[par * RCH + rw, q * 16:(q + 1) * 16]
                doff = (kv >> 14) - lo_v
                sv = kv & 16383
                mask = plsc.bitcast(doff, jnp.uint32) < jnp.uint32(R)
                plsc.store_compressed(
                    dbuf2.at[pl.ds(fp * BK + ptr, 16)], doff, mask=mask)
                plsc.store_compressed(
                    sbuf2.at[pl.ds(fp * BK + ptr, 16)], sv, mask=mask)
                ptr = ptr + plsc.all_reduce_population_count(mask)[0]

            def fl(op):
                p, f, pf = op
                start_gather(f)
                wait_gather(1 - f)
                drain(1 - f, pf)
                return jnp.int32(0), 1 - f, p

            return lax.cond(ptr > FLUSH_AT, fl, lambda op: op,
                            (ptr, fp, pfill))

        return lax.fori_loop(0, RCH, row, carry)

    ptr, fp, pfill = lax.fori_loop(
        0, NCH, chunk, (jnp.int32(0), jnp.int32(0), jnp.int32(0)))
    wait_edges(jnp.int32(NCH & 1))

    start_gather(fp)
    wait_gather(1 - fp)
    drain(1 - fp, pfill)
    wait_gather(fp)
    drain(fp, ptr)

    half = R // 2
    for c in range(2):
        pltpu.sync_copy(b_hbm.at[pl.ds(lo + c * half, half)],
                        rows2.at[pl.ds(0, half)])

        def comb(r, _):
            row = c * half + r
            for f in range(D // 16):
                sl = pl.ds(f * 16, 16)
                m = m_v[row, sl]
                seen = m != NEG
                val = jnp.maximum(m - rows2[r, sl], 0.0)
                m_v[row, sl] = jnp.where(seen, val, 0.0)
            return 0
        lax.fori_loop(0, half, comb, 0)

    pltpu.sync_copy(m_v.at[pl.ds(0, R)], out_hbm.at[pl.ds(lo, R)])


def _segmax(a, b, keys):
    fn = functools.partial(
        pl.kernel,
        out_type=jax.ShapeDtypeStruct((NPAD, D), jnp.float32),
        mesh=plsc.VectorSubcoreMesh(core_axis_name="c", subcore_axis_name="s"),
        compiler_params=pltpu.CompilerParams(needs_layout_passes=False),
        scratch_types=[
            pltpu.VMEM((R + 1, D), jnp.float32),
            pltpu.VMEM((2 * RCH, 128), jnp.int32),
            pltpu.VMEM((2 * BK,), jnp.int32),
            pltpu.VMEM((2 * BK,), jnp.int32),
            pltpu.VMEM((2 * BK, D), jnp.float32),
            pltpu.SemaphoreType.DMA,
            pltpu.SemaphoreType.DMA,
        ],
    )(_sc_body)
    return fn(a, b, keys)


def kernel(x_locs, pos_locs, edge_index, W, b):
    wx = W[:D]
    wpp = jnp.zeros((8, D), jnp.float32).at[:3].set(W[D:])
    xp = jnp.zeros((NPAD, D), jnp.float32).at[:N_NODES].set(x_locs)
    pp = jnp.zeros((NPAD, 8), jnp.float32).at[:N_NODES, :3].set(pos_locs)
    a, bmat = _compute_ab(xp, pp, wx, wpp, b.reshape(1, D))
    packed = (edge_index[1] << 14) | edge_index[0]
    packed = jnp.full((NEP,), 16383 << 14, jnp.int32).at[:N_EDGES].set(packed)
    out = _segmax(a, bmat, packed.reshape(NEP // 128, 128))
    return out[:N_NODES]


# speedup vs baseline: 19.1780x; 19.1780x over previous
import functools

import jax
import jax.numpy as jnp
from jax import lax
from jax.experimental import pallas as pl
from jax.experimental.pallas import tpu as pltpu
from jax.experimental.pallas import tpu_sc as plsc

N_NODES = 10000
N_EDGES = 320000
D = 128

NC = 2
NS = 16
NW = NC * NS
NPAD = 10240
R = NPAD // NW
NEP = 327680
ECH = 32768
NCH = NEP // ECH
BK = 256
NEG = float("-inf")


def _ab_body(x_ref, p_ref, wx_ref, wp_ref, b_ref, a_ref, bout_ref):
    pb = jnp.dot(p_ref[:], wp_ref[:], preferred_element_type=jnp.float32)
    a_ref[:] = (
        jnp.dot(x_ref[:], wx_ref[:], preferred_element_type=jnp.float32)
        + pb
        + b_ref[:]
    )
    bout_ref[:] = pb


def _compute_ab(xp, pp, wx, wpp, b2):
    blk = 1280
    grid = NPAD // blk
    return pl.pallas_call(
        _ab_body,
        grid=(grid,),
        in_specs=[
            pl.BlockSpec((blk, D), lambda i: (i, 0)),
            pl.BlockSpec((blk, 8), lambda i: (i, 0)),
            pl.BlockSpec((D, D), lambda i: (0, 0)),
            pl.BlockSpec((8, D), lambda i: (0, 0)),
            pl.BlockSpec((1, D), lambda i: (0, 0)),
        ],
        out_specs=[
            pl.BlockSpec((blk, D), lambda i: (i, 0)),
            pl.BlockSpec((blk, D), lambda i: (i, 0)),
        ],
        out_shape=[
            jax.ShapeDtypeStruct((NPAD, D), jnp.float32),
            jax.ShapeDtypeStruct((NPAD, D), jnp.float32),
        ],
    )(xp, pp, wx, wpp, b2)


def _sc_body(a_hbm, b_hbm, key_hbm, out_hbm,
             m_v, keych, sbuf, dbuf, rows, sem):
    cid = lax.axis_index("c")
    sid = lax.axis_index("s")
    wid = sid * NC + cid
    lo = wid * R
    lo_v = jnp.zeros((16,), jnp.int32) + lo

    neg = jnp.full((16,), NEG, jnp.float32)

    def init_row(i, _):
        for f in range(D // 16):
            m_v[i, f * 16:(f + 1) * 16] = neg
        return 0
    lax.fori_loop(0, R + 1, init_row, 0)

    zv = jnp.zeros((16,), jnp.int32)
    dumpv = jnp.full((16,), R, jnp.int32)
    for k16 in range(BK // 16):
        sl = pl.ds(k16 * 16, 16)
        sbuf[sl] = zv
        dbuf[sl] = dumpv

    def flush(p):
        pltpu.async_copy(a_hbm.at[sbuf], rows, sem).wait()

        def drain(k16, _):
            dvec = dbuf[pl.ds(k16 * 16, 16)]
            for j in range(16):
                r = dvec[j]
                k = k16 * 16 + j
                for f in range(D // 16):
                    sl = pl.ds(f * 16, 16)
                    m_v[r, sl] = jnp.maximum(m_v[r, sl], rows[k, sl])
            return 0
        lax.fori_loop(0, BK // 16, drain, 0)
        return jnp.int32(0)

    def group(g, ptr):
        kv = keych[pl.ds(g * 16, 16)]
        doff = (kv >> 14) - lo_v
        mask = plsc.bitcast(doff, jnp.uint32) < jnp.uint32(R)
        cnt = plsc.all_reduce_population_count(mask)[0]

        def has(p):
            sv = kv & 16383
            plsc.store_compressed(dbuf.at[pl.ds(p, 16)], doff, mask=mask)
            plsc.store_compressed(sbuf.at[pl.ds(p, 16)], sv, mask=mask)
            return p + cnt

        ptr = lax.cond(cnt > 0, has, lambda p: p, ptr)
        ptr = lax.cond(ptr > BK - 16, flush, lambda p: p, ptr)
        return ptr

    def chunk(c, ptr):
        pltpu.sync_copy(key_hbm.at[pl.ds(c * ECH, ECH)], keych)
        return lax.fori_loop(0, ECH // 16, group, ptr)

    ptr = lax.fori_loop(0, NCH, chunk, jnp.int32(0))
    flush(ptr)

    half = R // 2
    for c in range(2):
        pltpu.sync_copy(b_hbm.at[pl.ds(lo + c * half, half)],
                        rows.at[pl.ds(0, half)])

        def comb(r, _):
            row = c * half + r
            for f in range(D // 16):
                sl = pl.ds(f * 16, 16)
                m = m_v[row, sl]
                seen = m != NEG
                val = jnp.maximum(m - rows[r, sl], 0.0)
                m_v[row, sl] = jnp.where(seen, val, 0.0)
            return 0
        lax.fori_loop(0, half, comb, 0)

    pltpu.sync_copy(m_v.at[pl.ds(0, R)], out_hbm.at[pl.ds(lo, R)])


def _segmax(a, b, keys):
    fn = functools.partial(
        pl.kernel,
        out_type=jax.ShapeDtypeStruct((NPAD, D), jnp.float32),
        mesh=plsc.VectorSubcoreMesh(core_axis_name="c", subcore_axis_name="s"),
        compiler_params=pltpu.CompilerParams(needs_layout_passes=False),
        scratch_types=[
            pltpu.VMEM((R + 1, D), jnp.float32),
            pltpu.VMEM((ECH,), jnp.int32),
            pltpu.VMEM((BK,), jnp.int32),
            pltpu.VMEM((BK,), jnp.int32),
            pltpu.VMEM((BK, D), jnp.float32),
            pltpu.SemaphoreType.DMA,
        ],
    )(_sc_body)
    return fn(a, b, keys)


def kernel(x_locs, pos_locs, edge_index, W, b):
    wx = W[:D]
    wpp = jnp.zeros((8, D), jnp.float32).at[:3].set(W[D:])
    xp = jnp.zeros((NPAD, D), jnp.float32).at[:N_NODES].set(x_locs)
    pp = jnp.zeros((NPAD, 8), jnp.float32).at[:N_NODES, :3].set(pos_locs)
    a, bmat = _compute_ab(xp, pp, wx, wpp, b.reshape(1, D))
    packed = (edge_index[1] << 14) | edge_index[0]
    packed = jnp.full((NEP,), 16383 << 14, jnp.int32).at[:N_EDGES].set(packed)
    out = _segmax(a, bmat, packed)
    return out[:N_NODES]
